# baseline (speedup 1.0000x reference)
import functools

import jax
import jax.numpy as jnp
from jax import lax
from jax.experimental import pallas as pl
from jax.experimental.pallas import tpu as pltpu
from jax.experimental.pallas import tpu_sc as plsc

N = 4096
D = 256
R_PAD = 1024
REP = 8
E = 65536
E2 = 2 * E
KEY_LIMIT = E2 * R_PAD


def _scores_body(relt_ref, w_ref, out_ref):
    s = jnp.sum(relt_ref[...] * w_ref[...], axis=0, keepdims=True)
    out_ref[...] = jnp.broadcast_to(jnp.maximum(s, 0.0) + 1.0, (REP, R_PAD))


def _compute_scores(relt_pad, w_col):
    return pl.pallas_call(
        _scores_body,
        out_shape=jax.ShapeDtypeStruct((REP, R_PAD), jnp.float32),
    )(relt_pad, w_col)


_NC = 2
_NS = 16
_NW = _NC * _NS
_WPW = E2 // _NW
_CH = 128
_NCHUNK = _WPW // _CH

_sc_mesh = plsc.VectorSubcoreMesh(core_axis_name="c", subcore_axis_name="s")

_SCRATCH = [
    pltpu.VMEM((_NCHUNK, _CH), jnp.int32),
    pltpu.VMEM((_NCHUNK, _CH), jnp.int32),
    pltpu.VMEM((_NCHUNK, _CH), jnp.int32),
    pltpu.VMEM((_NCHUNK, _CH), jnp.int32),
    pltpu.VMEM((_NCHUNK, _CH), jnp.int32),
    pltpu.VMEM((_NCHUNK, _CH), jnp.int32),
    pltpu.VMEM((_NCHUNK, _CH), jnp.int32),
    pltpu.VMEM((_NCHUNK, _CH), jnp.int32),
    pltpu.VMEM((_NCHUNK, _CH), jnp.int32),
    pltpu.VMEM((_NCHUNK, _CH), jnp.int32),
    pltpu.SemaphoreType.DMA,
    pltpu.SemaphoreType.DMA,
]


def _stage(rows_hbm, cols_hbm, ridx_hbm, rows_v, cols_v, ridx_v, flat_v):
    wid = lax.axis_index("s") * _NC + lax.axis_index("c")
    rbase = wid * _NCHUNK
    pltpu.sync_copy(rows_hbm.at[pl.ds(rbase, _NCHUNK)], rows_v)
    pltpu.sync_copy(cols_hbm.at[pl.ds(rbase, _NCHUNK)], cols_v)
    pltpu.sync_copy(ridx_hbm.at[pl.ds(rbase, _NCHUNK)], ridx_v)
    for c in range(_NCHUNK):
        @pl.loop(0, _CH // 16)
        def _flat_body(j, c=c):
            sl = pl.ds(j * 16, 16)
            flat_v[c, sl] = rows_v[c, sl] * N + cols_v[c, sl]
    return wid


def _fill_keys(wid, ridx_v, key_v, xor_mask):
    base = wid * _WPW
    lane = lax.iota(jnp.int32, 16)
    for c in range(_NCHUNK):
        @pl.loop(0, _CH // 16)
        def _key_body(j, c=c):
            sl = pl.ds(j * 16, 16)
            pos = (base + c * _CH + j * 16 + lane) ^ xor_mask
            key_v[c, sl] = pos * R_PAD + ridx_v[c, sl]


def _make_keys_kernel(xor_mask):
    @functools.partial(pl.kernel, mesh=_sc_mesh, out_type=(),
                       scratch_types=_SCRATCH)
    def _keys(rows_hbm, cols_hbm, ridx_hbm, l_ref,
              rows_v, cols_v, ridx_v, flat_v, key_v, g1_v, g2_v, widx_v,
              t1_v, t2_v, sem, sem2):
        wid = _stage(rows_hbm, cols_hbm, ridx_hbm,
                     rows_v, cols_v, ridx_v, flat_v)
        _fill_keys(wid, ridx_v, key_v, xor_mask)
        scatters = [
            pltpu.async_copy(key_v.at[c], l_ref.at[flat_v.at[c]], sem)
            for c in range(_NCHUNK)
        ]
        for cp in scatters:
            cp.wait()

    return _keys


def _make_resolve_kernel(xor_mask):
    @functools.partial(pl.kernel, mesh=_sc_mesh, out_type=(),
                       scratch_types=_SCRATCH)
    def _resolve(rows_hbm, cols_hbm, ridx_hbm, l_ref,
                 rows_v, cols_v, ridx_v, flat_v, key_v, g1_v, g2_v, widx_v,
                 t1_v, t2_v, sem, sem2):
        wid = _stage(rows_hbm, cols_hbm, ridx_hbm,
                     rows_v, cols_v, ridx_v, flat_v)
        gathers = [
            pltpu.async_copy(l_ref.at[flat_v.at[c]], g1_v.at[c], sem)
            for c in range(_NCHUNK)
        ]
        for cp in gathers:
            cp.wait()
        _fill_keys(wid, ridx_v, key_v, xor_mask)
        lane = lax.iota(jnp.int32, 16)
        for c in range(_NCHUNK):
            @pl.loop(0, _CH // 16)
            def _mask_body(j, c=c):
                sl = pl.ds(j * 16, 16)
                win = key_v[c, sl] > g1_v[c, sl]
                dump = N * N + c * _CH + j * 16 + lane
                flat_v[c, sl] = jnp.where(win, flat_v[c, sl], dump)
        scatters = [
            pltpu.async_copy(key_v.at[c], l_ref.at[flat_v.at[c]], sem2)
            for c in range(_NCHUNK)
        ]
        for cp in scatters:
            cp.wait()

    return _resolve


_sc_keys_a = _make_keys_kernel(0)
_sc_keys_b = _make_keys_kernel(E - 1)
_sc_resolve_a = _make_resolve_kernel(0)
_sc_resolve_b = _make_resolve_kernel(E - 1)


@functools.partial(pl.kernel, mesh=_sc_mesh, out_type=(),
                   scratch_types=_SCRATCH)
def _sc_values(rows_hbm, cols_hbm, ridx_hbm, table_hbm, l_ref, l2_ref,
               rows_v, cols_v, ridx_v, flat_v, key_v, g1_v, g2_v, widx_v,
               t1_v, t2_v, sem, sem2):
    _stage(rows_hbm, cols_hbm, ridx_hbm, rows_v, cols_v, ridx_v, flat_v)
    gathers = [
        pltpu.async_copy(l_ref.at[flat_v.at[c]], g1_v.at[c], sem)
        for c in range(_NCHUNK)
    ] + [
        pltpu.async_copy(l2_ref.at[flat_v.at[c]], g2_v.at[c], sem)
        for c in range(_NCHUNK)
    ]
    for cp in gathers:
        cp.wait()
    spread = (lax.iota(jnp.int32, 16) & (REP - 1)) * R_PAD
    for c in range(_NCHUNK):
        @pl.loop(0, _CH // 16)
        def _widx_body(j, c=c):
            sl = pl.ds(j * 16, 16)
            widx_v[c, sl] = (g1_v[c, sl] & (R_PAD - 1)) + spread
            g2_v[c, sl] = (g2_v[c, sl] & (R_PAD - 1)) + spread
    tgathers = [
        pltpu.async_copy(table_hbm.at[widx_v.at[c]], t1_v.at[c], sem2)
        for c in range(_NCHUNK)
    ] + [
        pltpu.async_copy(table_hbm.at[g2_v.at[c]], t2_v.at[c], sem2)
        for c in range(_NCHUNK)
    ]
    for cp in tgathers:
        cp.wait()
    for c in range(_NCHUNK):
        @pl.loop(0, _CH // 16)
        def _val_body(j, c=c):
            sl = pl.ds(j * 16, 16)
            g = g1_v[c, sl]
            t1 = t1_v[c, sl]
            t2 = t2_v[c, sl]
            mid = (t1 >> 1) + (t2 >> 1) + (t1 & t2 & 1)
            key_v[c, sl] = jnp.where(g < KEY_LIMIT, mid, g)
    scatters = [
        pltpu.async_copy(key_v.at[c], l_ref.at[flat_v.at[c]], sem)
        for c in range(_NCHUNK)
    ]
    for cp in scatters:
        cp.wait()


_BR = 256


def _flash_body(li_ref, adj_ref, inp_ref, bias_ref, out_ref):
    lb = lax.bitcast_convert_type(li_ref[...], jnp.float32)
    l = jnp.where(lb > 0.5, lb - 1.0, 0.0)
    m = l + adj_ref[...]
    mx = jnp.max(m, axis=1, keepdims=True)
    e = jnp.exp(m - mx)
    z = jnp.sum(e, axis=1, keepdims=True)
    acc = jnp.dot(e, inp_ref[...], preferred_element_type=jnp.float32)
    r = acc / z + bias_ref[...]
    out_ref[...] = jnp.where(r > 0.0, r, jnp.exp(r) - 1.0)


def _flash_call(lmat, adj, inp, bias_row):
    grid = (N // _BR,)
    return pl.pallas_call(
        _flash_body,
        grid=grid,
        in_specs=[
            pl.BlockSpec((_BR, N), lambda i: (i, 0)),
            pl.BlockSpec((_BR, N), lambda i: (i, 0)),
            pl.BlockSpec((N, D), lambda i: (0, 0)),
            pl.BlockSpec((1, D), lambda i: (0, 0)),
        ],
        out_specs=pl.BlockSpec((_BR, D), lambda i: (i, 0)),
        out_shape=jax.ShapeDtypeStruct((N, D), jnp.float32),
    )(lmat, adj, inp, bias_row)


def kernel(input, rel, edge_index, rel_idx, adj, w_rel, bias):
    relt_pad = jnp.pad(rel, ((0, R_PAD - rel.shape[0]), (0, 0))).T
    table = _compute_scores(relt_pad, w_rel.reshape(-1, 1)).reshape(REP * R_PAD)
    table_bits = lax.bitcast_convert_type(table, jnp.int32)
    e1 = edge_index[0]
    e2 = edge_index[1]
    rows = jnp.concatenate([e1, e2]).reshape(E2 // _CH, _CH)
    cols = jnp.concatenate([e2, e1]).reshape(E2 // _CH, _CH)
    ridx = jnp.concatenate([rel_idx, rel_idx]).reshape(E2 // _CH, _CH)
    l_ref = jax.new_ref(jnp.zeros((N * (N + 1),), jnp.int32))
    l2_ref = jax.new_ref(jnp.zeros((N * (N + 1),), jnp.int32))
    _sc_keys_a(rows, cols, ridx, l_ref)
    _sc_keys_b(rows, cols, ridx, l2_ref)
    _sc_resolve_a(rows, cols, ridx, l_ref)
    _sc_resolve_b(rows, cols, ridx, l2_ref)
    _sc_resolve_a(rows, cols, ridx, l_ref)
    _sc_resolve_b(rows, cols, ridx, l2_ref)
    _sc_values(rows, cols, ridx, table_bits, l_ref, l2_ref)
    lmat = l_ref[...].reshape(N + 1, N)
    return _flash_call(lmat, adj, input, bias.reshape(1, D))


# speedup vs baseline: 4.6059x; 4.6059x over previous
import functools

import jax
import jax.numpy as jnp
from jax import lax
from jax.experimental import pallas as pl
from jax.experimental.pallas import tpu as pltpu
from jax.experimental.pallas import tpu_sc as plsc

N = 4096
D = 256
R_PAD = 1024
REP = 8
E = 65536
E2 = 2 * E
KEY_LIMIT = E2 * R_PAD


def _scores_body(relt_ref, w_ref, out_ref):
    s = jnp.sum(relt_ref[...] * w_ref[...], axis=0, keepdims=True)
    out_ref[...] = jnp.broadcast_to(jnp.maximum(s, 0.0) + 1.0, (REP, R_PAD))


def _compute_scores(relt_pad, w_col):
    return pl.pallas_call(
        _scores_body,
        out_shape=jax.ShapeDtypeStruct((REP, R_PAD), jnp.float32),
    )(relt_pad, w_col)


_NC = 2
_NS = 16
_NW = _NC * _NS
_WPW = E2 // _NW
_CH = 128
_NCHUNK = _WPW // _CH

_sc_mesh = plsc.VectorSubcoreMesh(core_axis_name="c", subcore_axis_name="s")

_SCRATCH = [
    pltpu.VMEM((_NCHUNK, _CH), jnp.int32),
    pltpu.VMEM((_NCHUNK, _CH), jnp.int32),
    pltpu.VMEM((_NCHUNK, _CH), jnp.int32),
    pltpu.VMEM((_NCHUNK, _CH), jnp.int32),
    pltpu.VMEM((_NCHUNK, _CH), jnp.int32),
    pltpu.VMEM((_NCHUNK, _CH), jnp.int32),
    pltpu.VMEM((_NCHUNK, _CH), jnp.int32),
    pltpu.VMEM((_NCHUNK, _CH), jnp.int32),
    pltpu.VMEM((_NCHUNK, _CH), jnp.int32),
    pltpu.VMEM((_NCHUNK, _CH), jnp.int32),
    pltpu.SemaphoreType.DMA,
    pltpu.SemaphoreType.DMA,
]


def _stage(rows_hbm, cols_hbm, ridx_hbm, rows_v, cols_v, ridx_v, flat_v):
    wid = lax.axis_index("s") * _NC + lax.axis_index("c")
    rbase = wid * _NCHUNK
    pltpu.sync_copy(rows_hbm.at[pl.ds(rbase, _NCHUNK)], rows_v)
    pltpu.sync_copy(cols_hbm.at[pl.ds(rbase, _NCHUNK)], cols_v)
    pltpu.sync_copy(ridx_hbm.at[pl.ds(rbase, _NCHUNK)], ridx_v)
    for c in range(_NCHUNK):
        @pl.loop(0, _CH // 16)
        def _flat_body(j, c=c):
            sl = pl.ds(j * 16, 16)
            flat_v[c, sl] = rows_v[c, sl] * N + cols_v[c, sl]
    return wid


def _fill_keys(wid, ridx_v, key_v, xor_mask):
    base = wid * _WPW
    lane = lax.iota(jnp.int32, 16)
    for c in range(_NCHUNK):
        @pl.loop(0, _CH // 16)
        def _key_body(j, c=c):
            sl = pl.ds(j * 16, 16)
            pos = (base + c * _CH + j * 16 + lane) ^ xor_mask
            key_v[c, sl] = pos * R_PAD + ridx_v[c, sl]


def _make_keys_kernel(xor_mask):
    @functools.partial(pl.kernel, mesh=_sc_mesh, out_type=(),
                       scratch_types=_SCRATCH)
    def _keys(rows_hbm, cols_hbm, ridx_hbm, l_ref,
              rows_v, cols_v, ridx_v, flat_v, key_v, g1_v, g2_v, widx_v,
              t1_v, t2_v, sem, sem2):
        wid = _stage(rows_hbm, cols_hbm, ridx_hbm,
                     rows_v, cols_v, ridx_v, flat_v)
        _fill_keys(wid, ridx_v, key_v, xor_mask)
        scatters = [
            pltpu.async_copy(key_v.at[c], l_ref.at[flat_v.at[c]], sem)
            for c in range(_NCHUNK)
        ]
        for cp in scatters:
            cp.wait()

    return _keys


def _make_resolve_kernel(xor_mask):
    @functools.partial(pl.kernel, mesh=_sc_mesh, out_type=(),
                       scratch_types=_SCRATCH)
    def _resolve(rows_hbm, cols_hbm, ridx_hbm, l_ref,
                 rows_v, cols_v, ridx_v, flat_v, key_v, g1_v, g2_v, widx_v,
                 t1_v, t2_v, sem, sem2):
        wid = _stage(rows_hbm, cols_hbm, ridx_hbm,
                     rows_v, cols_v, ridx_v, flat_v)
        gathers = [
            pltpu.async_copy(l_ref.at[flat_v.at[c]], g1_v.at[c], sem)
            for c in range(_NCHUNK)
        ]
        for cp in gathers:
            cp.wait()
        _fill_keys(wid, ridx_v, key_v, xor_mask)
        lane = lax.iota(jnp.int32, 16)
        dump_base = N * N + wid * _WPW
        for c in range(_NCHUNK):
            @pl.loop(0, _CH // 16)
            def _mask_body(j, c=c):
                sl = pl.ds(j * 16, 16)
                win = key_v[c, sl] > g1_v[c, sl]
                dump = dump_base + c * _CH + j * 16 + lane
                flat_v[c, sl] = jnp.where(win, flat_v[c, sl], dump)
        scatters = [
            pltpu.async_copy(key_v.at[c], l_ref.at[flat_v.at[c]], sem2)
            for c in range(_NCHUNK)
        ]
        for cp in scatters:
            cp.wait()

    return _resolve


_sc_keys_a = _make_keys_kernel(0)
_sc_keys_b = _make_keys_kernel(E - 1)
_sc_resolve_a = _make_resolve_kernel(0)
_sc_resolve_b = _make_resolve_kernel(E - 1)


@functools.partial(pl.kernel, mesh=_sc_mesh, out_type=(),
                   scratch_types=_SCRATCH)
def _sc_values(rows_hbm, cols_hbm, ridx_hbm, table_hbm, l_ref, l2_ref,
               rows_v, cols_v, ridx_v, flat_v, key_v, g1_v, g2_v, widx_v,
               t1_v, t2_v, sem, sem2):
    _stage(rows_hbm, cols_hbm, ridx_hbm, rows_v, cols_v, ridx_v, flat_v)
    gathers = [
        pltpu.async_copy(l_ref.at[flat_v.at[c]], g1_v.at[c], sem)
        for c in range(_NCHUNK)
    ] + [
        pltpu.async_copy(l2_ref.at[flat_v.at[c]], g2_v.at[c], sem)
        for c in range(_NCHUNK)
    ]
    for cp in gathers:
        cp.wait()
    spread = (lax.iota(jnp.int32, 16) & (REP - 1)) * R_PAD
    for c in range(_NCHUNK):
        @pl.loop(0, _CH // 16)
        def _widx_body(j, c=c):
            sl = pl.ds(j * 16, 16)
            widx_v[c, sl] = (g1_v[c, sl] & (R_PAD - 1)) + spread
            g2_v[c, sl] = (g2_v[c, sl] & (R_PAD - 1)) + spread
    tgathers = [
        pltpu.async_copy(table_hbm.at[widx_v.at[c]], t1_v.at[c], sem2)
        for c in range(_NCHUNK)
    ] + [
        pltpu.async_copy(table_hbm.at[g2_v.at[c]], t2_v.at[c], sem2)
        for c in range(_NCHUNK)
    ]
    for cp in tgathers:
        cp.wait()
    for c in range(_NCHUNK):
        @pl.loop(0, _CH // 16)
        def _val_body(j, c=c):
            sl = pl.ds(j * 16, 16)
            g = g1_v[c, sl]
            t1 = t1_v[c, sl]
            t2 = t2_v[c, sl]
            mid = (t1 >> 1) + (t2 >> 1) + (t1 & t2 & 1)
            key_v[c, sl] = jnp.where(g < KEY_LIMIT, mid, g)
    scatters = [
        pltpu.async_copy(key_v.at[c], l_ref.at[flat_v.at[c]], sem)
        for c in range(_NCHUNK)
    ]
    for cp in scatters:
        cp.wait()


_BR = 256


def _flash_body(li_ref, adj_ref, inp_ref, bias_ref, out_ref):
    lb = lax.bitcast_convert_type(li_ref[...], jnp.float32)
    l = jnp.where(lb > 0.5, lb - 1.0, 0.0)
    m = l + adj_ref[...]
    mx = jnp.max(m, axis=1, keepdims=True)
    e = jnp.exp(m - mx)
    z = jnp.sum(e, axis=1, keepdims=True)
    acc = jnp.dot(e, inp_ref[...], preferred_element_type=jnp.float32)
    r = acc / z + bias_ref[...]
    out_ref[...] = jnp.where(r > 0.0, r, jnp.exp(r) - 1.0)


def _flash_call(lmat, adj, inp, bias_row):
    grid = (N // _BR,)
    return pl.pallas_call(
        _flash_body,
        grid=grid,
        in_specs=[
            pl.BlockSpec((_BR, N), lambda i: (i, 0)),
            pl.BlockSpec((_BR, N), lambda i: (i, 0)),
            pl.BlockSpec((N, D), lambda i: (0, 0)),
            pl.BlockSpec((1, D), lambda i: (0, 0)),
        ],
        out_specs=pl.BlockSpec((_BR, D), lambda i: (i, 0)),
        out_shape=jax.ShapeDtypeStruct((N, D), jnp.float32),
    )(lmat, adj, inp, bias_row)


def kernel(input, rel, edge_index, rel_idx, adj, w_rel, bias):
    relt_pad = jnp.pad(rel, ((0, R_PAD - rel.shape[0]), (0, 0))).T
    table = _compute_scores(relt_pad, w_rel.reshape(-1, 1)).reshape(REP * R_PAD)
    table_bits = lax.bitcast_convert_type(table, jnp.int32)
    e1 = edge_index[0]
    e2 = edge_index[1]
    rows = jnp.concatenate([e1, e2]).reshape(E2 // _CH, _CH)
    cols = jnp.concatenate([e2, e1]).reshape(E2 // _CH, _CH)
    ridx = jnp.concatenate([rel_idx, rel_idx]).reshape(E2 // _CH, _CH)
    l_ref = jax.new_ref(jnp.zeros((N * (N + _NW),), jnp.int32))
    l2_ref = jax.new_ref(jnp.zeros((N * (N + _NW),), jnp.int32))
    _sc_keys_a(rows, cols, ridx, l_ref)
    _sc_keys_b(rows, cols, ridx, l2_ref)
    _sc_resolve_a(rows, cols, ridx, l_ref)
    _sc_resolve_b(rows, cols, ridx, l2_ref)
    _sc_values(rows, cols, ridx, table_bits, l_ref, l2_ref)
    lmat = l_ref[...].reshape(N + _NW, N)
    return _flash_call(lmat, adj, input, bias.reshape(1, D))


# speedup vs baseline: 4.6543x; 1.0105x over previous
import functools

import jax
import jax.numpy as jnp
from jax import lax
from jax.experimental import pallas as pl
from jax.experimental.pallas import tpu as pltpu
from jax.experimental.pallas import tpu_sc as plsc

N = 4096
D = 256
R_PAD = 1024
REP = 8
E = 65536
E2 = 2 * E
KEY_LIMIT = E2 * R_PAD


def _scores_body(relt_ref, w_ref, out_ref):
    s = jnp.sum(relt_ref[...] * w_ref[...], axis=0, keepdims=True)
    out_ref[...] = jnp.broadcast_to(jnp.maximum(s, 0.0) + 1.0, (REP, R_PAD))


def _compute_scores(relt_pad, w_col):
    return pl.pallas_call(
        _scores_body,
        out_shape=jax.ShapeDtypeStruct((REP, R_PAD), jnp.float32),
    )(relt_pad, w_col)


_NC = 2
_NS = 16
_NW = _NC * _NS
_WPW = E2 // _NW
_NV = _WPW // 16

_sc_mesh = plsc.VectorSubcoreMesh(core_axis_name="c", subcore_axis_name="s")

_SCRATCH = [
    pltpu.VMEM((_WPW,), jnp.int32),
    pltpu.VMEM((_WPW,), jnp.int32),
    pltpu.VMEM((_WPW,), jnp.int32),
    pltpu.VMEM((_WPW,), jnp.int32),
    pltpu.VMEM((_WPW,), jnp.int32),
    pltpu.VMEM((_WPW,), jnp.int32),
    pltpu.VMEM((_WPW,), jnp.int32),
    pltpu.VMEM((_WPW,), jnp.int32),
    pltpu.VMEM((_WPW,), jnp.int32),
    pltpu.VMEM((_WPW,), jnp.int32),
    pltpu.SemaphoreType.DMA,
    pltpu.SemaphoreType.DMA,
]


def _stage(rows_hbm, cols_hbm, ridx_hbm, rows_v, cols_v, ridx_v, flat_v):
    wid = lax.axis_index("s") * _NC + lax.axis_index("c")
    pltpu.sync_copy(rows_hbm.at[wid], rows_v)
    pltpu.sync_copy(cols_hbm.at[wid], cols_v)
    pltpu.sync_copy(ridx_hbm.at[wid], ridx_v)

    @pl.loop(0, _NV)
    def _flat_body(j):
        sl = pl.ds(j * 16, 16)
        flat_v[sl] = rows_v[sl] * N + cols_v[sl]

    return wid


def _fill_keys(wid, ridx_v, key_v, xor_mask):
    base = wid * _WPW
    lane = lax.iota(jnp.int32, 16)

    @pl.loop(0, _NV)
    def _key_body(j):
        sl = pl.ds(j * 16, 16)
        pos = (base + j * 16 + lane) ^ xor_mask
        key_v[sl] = pos * R_PAD + ridx_v[sl]


def _make_keys_kernel(xor_mask):
    @functools.partial(pl.kernel, mesh=_sc_mesh, out_type=(),
                       scratch_types=_SCRATCH)
    def _keys(rows_hbm, cols_hbm, ridx_hbm, l_ref,
              rows_v, cols_v, ridx_v, flat_v, key_v, g1_v, g2_v, widx_v,
              t1_v, t2_v, sem, sem2):
        wid = _stage(rows_hbm, cols_hbm, ridx_hbm,
                     rows_v, cols_v, ridx_v, flat_v)
        _fill_keys(wid, ridx_v, key_v, xor_mask)
        pltpu.async_copy(key_v, l_ref.at[flat_v], sem).wait()

    return _keys


def _make_resolve_kernel(xor_mask):
    @functools.partial(pl.kernel, mesh=_sc_mesh, out_type=(),
                       scratch_types=_SCRATCH)
    def _resolve(rows_hbm, cols_hbm, ridx_hbm, l_ref,
                 rows_v, cols_v, ridx_v, flat_v, key_v, g1_v, g2_v, widx_v,
                 t1_v, t2_v, sem, sem2):
        wid = _stage(rows_hbm, cols_hbm, ridx_hbm,
                     rows_v, cols_v, ridx_v, flat_v)
        pltpu.async_copy(l_ref.at[flat_v], g1_v, sem).wait()
        _fill_keys(wid, ridx_v, key_v, xor_mask)
        lane = lax.iota(jnp.int32, 16)
        dump_base = N * N + wid * _WPW

        @pl.loop(0, _NV)
        def _mask_body(j):
            sl = pl.ds(j * 16, 16)
            win = key_v[sl] > g1_v[sl]
            dump = dump_base + j * 16 + lane
            flat_v[sl] = jnp.where(win, flat_v[sl], dump)

        pltpu.async_copy(key_v, l_ref.at[flat_v], sem2).wait()

    return _resolve


_sc_keys_a = _make_keys_kernel(0)
_sc_keys_b = _make_keys_kernel(E - 1)
_sc_resolve_a = _make_resolve_kernel(0)
_sc_resolve_b = _make_resolve_kernel(E - 1)


@functools.partial(pl.kernel, mesh=_sc_mesh, out_type=(),
                   scratch_types=_SCRATCH)
def _sc_values(rows_hbm, cols_hbm, ridx_hbm, table_hbm, l_ref, l2_ref,
               rows_v, cols_v, ridx_v, flat_v, key_v, g1_v, g2_v, widx_v,
               t1_v, t2_v, sem, sem2):
    _stage(rows_hbm, cols_hbm, ridx_hbm, rows_v, cols_v, ridx_v, flat_v)
    cp1 = pltpu.async_copy(l_ref.at[flat_v], g1_v, sem)
    cp2 = pltpu.async_copy(l2_ref.at[flat_v], g2_v, sem)
    cp1.wait()
    cp2.wait()
    spread = (lax.iota(jnp.int32, 16) & (REP - 1)) * R_PAD

    @pl.loop(0, _NV)
    def _widx_body(j):
        sl = pl.ds(j * 16, 16)
        widx_v[sl] = (g1_v[sl] & (R_PAD - 1)) + spread
        g2_v[sl] = (g2_v[sl] & (R_PAD - 1)) + spread

    cp3 = pltpu.async_copy(table_hbm.at[widx_v], t1_v, sem2)
    cp4 = pltpu.async_copy(table_hbm.at[g2_v], t2_v, sem2)
    cp3.wait()
    cp4.wait()

    @pl.loop(0, _NV)
    def _val_body(j):
        sl = pl.ds(j * 16, 16)
        g = g1_v[sl]
        t1 = t1_v[sl]
        t2 = t2_v[sl]
        mid = (t1 >> 1) + (t2 >> 1) + (t1 & t2 & 1)
        key_v[sl] = jnp.where(g < KEY_LIMIT, mid, g)

    pltpu.async_copy(key_v, l_ref.at[flat_v], sem).wait()


_BR = 256


def _flash_body(li_ref, adj_ref, inp_ref, bias_ref, out_ref):
    lb = lax.bitcast_convert_type(li_ref[...], jnp.float32)
    l = jnp.where(lb > 0.5, lb - 1.0, 0.0)
    m = l + adj_ref[...]
    mx = jnp.max(m, axis=1, keepdims=True)
    e = jnp.exp(m - mx)
    z = jnp.sum(e, axis=1, keepdims=True)
    acc = jnp.dot(e, inp_ref[...], preferred_element_type=jnp.float32)
    r = acc / z + bias_ref[...]
    out_ref[...] = jnp.where(r > 0.0, r, jnp.exp(r) - 1.0)


def _flash_call(lmat, adj, inp, bias_row):
    grid = (N // _BR,)
    return pl.pallas_call(
        _flash_body,
        grid=grid,
        in_specs=[
            pl.BlockSpec((_BR, N), lambda i: (i, 0)),
            pl.BlockSpec((_BR, N), lambda i: (i, 0)),
            pl.BlockSpec((N, D), lambda i: (0, 0)),
            pl.BlockSpec((1, D), lambda i: (0, 0)),
        ],
        out_specs=pl.BlockSpec((_BR, D), lambda i: (i, 0)),
        out_shape=jax.ShapeDtypeStruct((N, D), jnp.float32),
    )(lmat, adj, inp, bias_row)


def kernel(input, rel, edge_index, rel_idx, adj, w_rel, bias):
    relt_pad = jnp.pad(rel, ((0, R_PAD - rel.shape[0]), (0, 0))).T
    table = _compute_scores(relt_pad, w_rel.reshape(-1, 1)).reshape(REP * R_PAD)
    table_bits = lax.bitcast_convert_type(table, jnp.int32)
    e1 = edge_index[0]
    e2 = edge_index[1]
    rows = jnp.concatenate([e1, e2]).reshape(_NW, _WPW)
    cols = jnp.concatenate([e2, e1]).reshape(_NW, _WPW)
    ridx = jnp.concatenate([rel_idx, rel_idx]).reshape(_NW, _WPW)
    l_ref = jax.new_ref(jnp.zeros((N * (N + _NW),), jnp.int32))
    l2_ref = jax.new_ref(jnp.zeros((N * (N + _NW),), jnp.int32))
    _sc_keys_a(rows, cols, ridx, l_ref)
    _sc_keys_b(rows, cols, ridx, l2_ref)
    _sc_resolve_a(rows, cols, ridx, l_ref)
    _sc_resolve_b(rows, cols, ridx, l2_ref)
    _sc_values(rows, cols, ridx, table_bits, l_ref, l2_ref)
    lmat = l_ref[...].reshape(N + _NW, N)
    return _flash_call(lmat, adj, input, bias.reshape(1, D))


# speedup vs baseline: 8.7709x; 1.8845x over previous
import functools

import jax
import jax.numpy as jnp
from jax import lax
from jax.experimental import pallas as pl
from jax.experimental.pallas import tpu as pltpu
from jax.experimental.pallas import tpu_sc as plsc

N = 4096
D = 256
R_PAD = 1024
REP = 8
E = 65536


def _scores_body(relt_ref, w_ref, out_ref):
    s = jnp.sum(relt_ref[...] * w_ref[...], axis=0, keepdims=True)
    out_ref[...] = jnp.broadcast_to(jnp.maximum(s, 0.0) + 1.0, (REP, R_PAD))


def _compute_scores(relt_pad, w_col):
    return pl.pallas_call(
        _scores_body,
        out_shape=jax.ShapeDtypeStruct((REP, R_PAD), jnp.float32),
    )(relt_pad, w_col)


_NC = 2
_NS = 16
_NW = _NC * _NS
_EPW = E // _NW
_E2PW = 2 * _EPW
_DUMP_ROWS = (2 * E + N - 1) // N

_sc_mesh = plsc.VectorSubcoreMesh(core_axis_name="c", subcore_axis_name="s")

_SCATTER_SCRATCH = [
    pltpu.VMEM((_EPW,), jnp.int32),
    pltpu.VMEM((_EPW,), jnp.int32),
    pltpu.VMEM((_EPW,), jnp.int32),
    pltpu.VMEM((_EPW,), jnp.int32),
    pltpu.VMEM((_EPW,), jnp.float32),
    pltpu.SemaphoreType.DMA,
]


def _make_phase_kernel(sign):
    @functools.partial(pl.kernel, mesh=_sc_mesh, out_type=(),
                       scratch_types=_SCATTER_SCRATCH)
    def _phase(rows_hbm, cols_hbm, ridx_hbm, table_hbm, buf_ref,
               r_v, c_v, ri_v, flat_v, val_v, sem):
        wid = lax.axis_index("s") * _NC + lax.axis_index("c")
        pltpu.sync_copy(rows_hbm.at[wid], r_v)
        pltpu.sync_copy(cols_hbm.at[wid], c_v)
        pltpu.sync_copy(ridx_hbm.at[wid], ri_v)
        spread = (lax.iota(jnp.int32, 16) & (REP - 1)) * R_PAD

        @pl.loop(0, _EPW // 16)
        def _prep(j):
            sl = pl.ds(j * 16, 16)
            flat_v[sl] = r_v[sl] * N + c_v[sl]
            ri_v[sl] = ri_v[sl] + spread

        pltpu.async_copy(table_hbm.at[ri_v], val_v, sem).wait()

        @pl.loop(0, _EPW // 16)
        def _sgn(j):
            sl = pl.ds(j * 16, 16)
            val_v[sl] = val_v[sl] * sign

        pltpu.async_copy(val_v, buf_ref.at[flat_v], sem).wait()

    return _phase


_sc_fwd = _make_phase_kernel(-1.0)
_sc_bwd = _make_phase_kernel(1.0)

_CHK_SCRATCH = [
    pltpu.VMEM((_E2PW,), jnp.int32),
    pltpu.VMEM((_E2PW,), jnp.float32),
    pltpu.VMEM((_E2PW,), jnp.float32),
    pltpu.VMEM((_E2PW,), jnp.int32),
    pltpu.VMEM((_E2PW,), jnp.float32),
    pltpu.VMEM((_EPW,), jnp.int32),
    pltpu.VMEM((_EPW,), jnp.int32),
    pltpu.VMEM((_EPW,), jnp.int32),
    pltpu.SemaphoreType.DMA,
]


@functools.partial(pl.kernel, mesh=_sc_mesh, out_type=(),
                   scratch_types=_CHK_SCRATCH)
def _sc_check(rows_hbm, cols_hbm, ridx_hbm, table_hbm, buf_ref,
              pflat_hbm, pval_hbm,
              flat_v, own_v, got_v, pt_v, pv_v, a_v, b_v, ri_v, sem):
    wid = lax.axis_index("s") * _NC + lax.axis_index("c")
    pltpu.sync_copy(rows_hbm.at[wid], a_v)
    pltpu.sync_copy(cols_hbm.at[wid], b_v)
    pltpu.sync_copy(ridx_hbm.at[wid], ri_v)
    spread = (lax.iota(jnp.int32, 16) & (REP - 1)) * R_PAD

    @pl.loop(0, _EPW // 16)
    def _prep(j):
        sl = pl.ds(j * 16, 16)
        sl2 = pl.ds(_EPW + j * 16, 16)
        flat_v[sl] = a_v[sl] * N + b_v[sl]
        flat_v[sl2] = b_v[sl] * N + a_v[sl]
        ri_v[sl] = ri_v[sl] + spread

    cpg = pltpu.async_copy(table_hbm.at[ri_v], got_v.at[pl.ds(0, _EPW)], sem)
    cpg.wait()

    @pl.loop(0, _EPW // 16)
    def _own(j):
        sl = pl.ds(j * 16, 16)
        sl2 = pl.ds(_EPW + j * 16, 16)
        t = got_v[sl]
        own_v[sl] = -t
        own_v[sl2] = t

    pltpu.async_copy(buf_ref.at[flat_v], got_v, sem).wait()

    lane = lax.iota(jnp.int32, 16)
    dump = N * N + wid * _E2PW

    @pl.loop(0, _E2PW // 16)
    def _decide(j):
        sl = pl.ds(j * 16, 16)
        g = got_v[sl]
        mine = own_v[sl]
        same_phase = jnp.where(mine < 0.0, g < 0.0, g > 0.0)
        patch = same_phase & (g != mine)
        pt_v[sl] = jnp.where(patch, flat_v[sl], dump + j * 16 + lane)
        pv_v[sl] = (g + mine) * 0.5

    pltpu.sync_copy(pt_v, pflat_hbm.at[wid])
    pltpu.sync_copy(pv_v, pval_hbm.at[wid])


@functools.partial(pl.kernel, mesh=_sc_mesh, out_type=(),
                   scratch_types=[
                       pltpu.VMEM((_E2PW,), jnp.int32),
                       pltpu.VMEM((_E2PW,), jnp.float32),
                       pltpu.SemaphoreType.DMA,
                   ])
def _sc_patch(pflat_hbm, pval_hbm, buf_ref, pt_v, pv_v, sem):
    wid = lax.axis_index("s") * _NC + lax.axis_index("c")
    pltpu.sync_copy(pflat_hbm.at[wid], pt_v)
    pltpu.sync_copy(pval_hbm.at[wid], pv_v)
    pltpu.async_copy(pv_v, buf_ref.at[pt_v], sem).wait()


_BR = 256


def _flash_body(b_ref, adj_ref, inp_ref, bias_ref, out_ref):
    lb = b_ref[...]
    l = jnp.where(lb != 0.0, jnp.abs(lb) - 1.0, 0.0)
    m = l + adj_ref[...]
    mx = jnp.max(m, axis=1, keepdims=True)
    e = jnp.exp(m - mx)
    z = jnp.sum(e, axis=1, keepdims=True)
    acc = jnp.dot(e, inp_ref[...], preferred_element_type=jnp.float32)
    r = acc / z + bias_ref[...]
    out_ref[...] = jnp.where(r > 0.0, r, jnp.exp(r) - 1.0)


def _flash_call(buf, adj, inp, bias_row):
    grid = (N // _BR,)
    return pl.pallas_call(
        _flash_body,
        grid=grid,
        in_specs=[
            pl.BlockSpec((_BR, N), lambda i: (i, 0)),
            pl.BlockSpec((_BR, N), lambda i: (i, 0)),
            pl.BlockSpec((N, D), lambda i: (0, 0)),
            pl.BlockSpec((1, D), lambda i: (0, 0)),
        ],
        out_specs=pl.BlockSpec((_BR, D), lambda i: (i, 0)),
        out_shape=jax.ShapeDtypeStruct((N, D), jnp.float32),
    )(buf, adj, inp, bias_row)


def kernel(input, rel, edge_index, rel_idx, adj, w_rel, bias):
    relt_pad = jnp.pad(rel, ((0, R_PAD - rel.shape[0]), (0, 0))).T
    table = _compute_scores(relt_pad, w_rel.reshape(-1, 1)).reshape(REP * R_PAD)
    e1 = edge_index[0].reshape(_NW, _EPW)
    e2 = edge_index[1].reshape(_NW, _EPW)
    ridx = rel_idx.reshape(_NW, _EPW)
    buf_ref = jax.new_ref(jnp.zeros((N * (N + _DUMP_ROWS),), jnp.float32))
    pflat_ref = jax.new_ref(jnp.zeros((_NW, _E2PW), jnp.int32))
    pval_ref = jax.new_ref(jnp.zeros((_NW, _E2PW), jnp.float32))
    _sc_fwd(e1, e2, ridx, table, buf_ref)
    _sc_bwd(e2, e1, ridx, table, buf_ref)
    _sc_check(e1, e2, ridx, table, buf_ref, pflat_ref, pval_ref)
    _sc_patch(pflat_ref, pval_ref, buf_ref)
    buf = buf_ref[...].reshape(N + _DUMP_ROWS, N)
    return _flash_call(buf, adj, input, bias.reshape(1, D))


# speedup vs baseline: 8.8338x; 1.0072x over previous
import functools

import jax
import jax.numpy as jnp
from jax import lax
from jax.experimental import pallas as pl
from jax.experimental.pallas import tpu as pltpu
from jax.experimental.pallas import tpu_sc as plsc

N = 4096
D = 256
R_PAD = 1024
REP = 1
E = 65536


def _scores_body(relt_ref, w_ref, out_ref):
    s = jnp.sum(relt_ref[...] * w_ref[...], axis=0, keepdims=True)
    out_ref[...] = jnp.broadcast_to(jnp.maximum(s, 0.0) + 1.0, (REP, R_PAD))


def _compute_scores(relt_pad, w_col):
    return pl.pallas_call(
        _scores_body,
        out_shape=jax.ShapeDtypeStruct((REP, R_PAD), jnp.float32),
    )(relt_pad, w_col)


_NC = 2
_NS = 16
_NW = _NC * _NS
_EPW = E // _NW
_E2PW = 2 * _EPW
_DUMP_ROWS = (2 * E + N - 1) // N

_sc_mesh = plsc.VectorSubcoreMesh(core_axis_name="c", subcore_axis_name="s")

_SCATTER_SCRATCH = [
    pltpu.VMEM((_EPW,), jnp.int32),
    pltpu.VMEM((_EPW,), jnp.int32),
    pltpu.VMEM((_EPW,), jnp.int32),
    pltpu.VMEM((_EPW,), jnp.int32),
    pltpu.VMEM((_EPW,), jnp.float32),
    pltpu.VMEM_SHARED((R_PAD,), jnp.float32),
    pltpu.SemaphoreType.DMA,
]


def _stage_table(table_hbm, spmem_ref):
    @pl.when(lax.axis_index("s") == 0)
    def _():
        pltpu.sync_copy(table_hbm, spmem_ref)

    plsc.subcore_barrier()


def _make_phase_kernel(sign):
    @functools.partial(pl.kernel, mesh=_sc_mesh, out_type=(),
                       scratch_types=_SCATTER_SCRATCH)
    def _phase(rows_hbm, cols_hbm, ridx_hbm, table_hbm, buf_ref,
               r_v, c_v, ri_v, flat_v, val_v, tab_s, sem):
        wid = lax.axis_index("s") * _NC + lax.axis_index("c")
        _stage_table(table_hbm, tab_s)
        pltpu.sync_copy(rows_hbm.at[wid], r_v)
        pltpu.sync_copy(cols_hbm.at[wid], c_v)
        pltpu.sync_copy(ridx_hbm.at[wid], ri_v)

        @pl.loop(0, _EPW // 16)
        def _prep(j):
            sl = pl.ds(j * 16, 16)
            flat_v[sl] = r_v[sl] * N + c_v[sl]

        pltpu.async_copy(tab_s.at[ri_v], val_v, sem).wait()

        @pl.loop(0, _EPW // 16)
        def _sgn(j):
            sl = pl.ds(j * 16, 16)
            val_v[sl] = val_v[sl] * sign

        pltpu.async_copy(val_v, buf_ref.at[flat_v], sem).wait()

    return _phase


_sc_fwd = _make_phase_kernel(-1.0)
_sc_bwd = _make_phase_kernel(1.0)

_CHK_SCRATCH = [
    pltpu.VMEM((_E2PW,), jnp.int32),
    pltpu.VMEM((_E2PW,), jnp.float32),
    pltpu.VMEM((_E2PW,), jnp.float32),
    pltpu.VMEM((_E2PW,), jnp.int32),
    pltpu.VMEM((_E2PW,), jnp.float32),
    pltpu.VMEM((_EPW,), jnp.int32),
    pltpu.VMEM((_EPW,), jnp.int32),
    pltpu.VMEM((_EPW,), jnp.int32),
    pltpu.VMEM_SHARED((R_PAD,), jnp.float32),
    pltpu.SemaphoreType.DMA,
]


@functools.partial(pl.kernel, mesh=_sc_mesh, out_type=(),
                   scratch_types=_CHK_SCRATCH)
def _sc_check(rows_hbm, cols_hbm, ridx_hbm, table_hbm, buf_ref,
              pflat_hbm, pval_hbm,
              flat_v, own_v, got_v, pt_v, pv_v, a_v, b_v, ri_v, tab_s, sem):
    wid = lax.axis_index("s") * _NC + lax.axis_index("c")
    _stage_table(table_hbm, tab_s)
    pltpu.sync_copy(rows_hbm.at[wid], a_v)
    pltpu.sync_copy(cols_hbm.at[wid], b_v)
    pltpu.sync_copy(ridx_hbm.at[wid], ri_v)

    @pl.loop(0, _EPW // 16)
    def _prep(j):
        sl = pl.ds(j * 16, 16)
        sl2 = pl.ds(_EPW + j * 16, 16)
        flat_v[sl] = a_v[sl] * N + b_v[sl]
        flat_v[sl2] = b_v[sl] * N + a_v[sl]

    cpg = pltpu.async_copy(tab_s.at[ri_v], got_v.at[pl.ds(0, _EPW)], sem)
    cpg.wait()

    @pl.loop(0, _EPW // 16)
    def _own(j):
        sl = pl.ds(j * 16, 16)
        sl2 = pl.ds(_EPW + j * 16, 16)
        t = got_v[sl]
        own_v[sl] = -t
        own_v[sl2] = t

    pltpu.async_copy(buf_ref.at[flat_v], got_v, sem).wait()

    lane = lax.iota(jnp.int32, 16)
    dump = N * N + wid * _E2PW

    @pl.loop(0, _E2PW // 16)
    def _decide(j):
        sl = pl.ds(j * 16, 16)
        g = got_v[sl]
        mine = own_v[sl]
        same_phase = jnp.where(mine < 0.0, g < 0.0, g > 0.0)
        patch = same_phase & (g != mine)
        pt_v[sl] = jnp.where(patch, flat_v[sl], dump + j * 16 + lane)
        pv_v[sl] = (g + mine) * 0.5

    pltpu.sync_copy(pt_v, pflat_hbm.at[wid])
    pltpu.sync_copy(pv_v, pval_hbm.at[wid])


@functools.partial(pl.kernel, mesh=_sc_mesh, out_type=(),
                   scratch_types=[
                       pltpu.VMEM((_E2PW,), jnp.int32),
                       pltpu.VMEM((_E2PW,), jnp.float32),
                       pltpu.SemaphoreType.DMA,
                   ])
def _sc_patch(pflat_hbm, pval_hbm, buf_ref, pt_v, pv_v, sem):
    wid = lax.axis_index("s") * _NC + lax.axis_index("c")
    pltpu.sync_copy(pflat_hbm.at[wid], pt_v)
    pltpu.sync_copy(pval_hbm.at[wid], pv_v)
    pltpu.async_copy(pv_v, buf_ref.at[pt_v], sem).wait()


_BR = 256


def _flash_body(b_ref, adj_ref, inp_ref, bias_ref, out_ref):
    lb = b_ref[...]
    l = jnp.where(lb != 0.0, jnp.abs(lb) - 1.0, 0.0)
    m = l + adj_ref[...]
    mx = jnp.max(m, axis=1, keepdims=True)
    e = jnp.exp(m - mx)
    z = jnp.sum(e, axis=1, keepdims=True)
    acc = jnp.dot(e, inp_ref[...], preferred_element_type=jnp.float32)
    r = acc / z + bias_ref[...]
    out_ref[...] = jnp.where(r > 0.0, r, jnp.exp(r) - 1.0)


def _flash_call(buf, adj, inp, bias_row):
    grid = (N // _BR,)
    return pl.pallas_call(
        _flash_body,
        grid=grid,
        in_specs=[
            pl.BlockSpec((_BR, N), lambda i: (i, 0)),
            pl.BlockSpec((_BR, N), lambda i: (i, 0)),
            pl.BlockSpec((N, D), lambda i: (0, 0)),
            pl.BlockSpec((1, D), lambda i: (0, 0)),
        ],
        out_specs=pl.BlockSpec((_BR, D), lambda i: (i, 0)),
        out_shape=jax.ShapeDtypeStruct((N, D), jnp.float32),
    )(buf, adj, inp, bias_row)


def kernel(input, rel, edge_index, rel_idx, adj, w_rel, bias):
    relt_pad = jnp.pad(rel, ((0, R_PAD - rel.shape[0]), (0, 0))).T
    table = _compute_scores(relt_pad, w_rel.reshape(-1, 1)).reshape(REP * R_PAD)
    e1 = edge_index[0].reshape(_NW, _EPW)
    e2 = edge_index[1].reshape(_NW, _EPW)
    ridx = rel_idx.reshape(_NW, _EPW)
    buf_ref = jax.new_ref(jnp.zeros((N * (N + _DUMP_ROWS),), jnp.float32))
    pflat_ref = jax.new_ref(jnp.zeros((_NW, _E2PW), jnp.int32))
    pval_ref = jax.new_ref(jnp.zeros((_NW, _E2PW), jnp.float32))
    _sc_fwd(e1, e2, ridx, table, buf_ref)
    _sc_bwd(e2, e1, ridx, table, buf_ref)
    _sc_check(e1, e2, ridx, table, buf_ref, pflat_ref, pval_ref)
    _sc_patch(pflat_ref, pval_ref, buf_ref)
    buf = buf_ref[...].reshape(N + _DUMP_ROWS, N)
    return _flash_call(buf, adj, input, bias.reshape(1, D))


# speedup vs baseline: 8.8414x; 1.0009x over previous
import functools

import jax
import jax.numpy as jnp
from jax import lax
from jax.experimental import pallas as pl
from jax.experimental.pallas import tpu as pltpu
from jax.experimental.pallas import tpu_sc as plsc

N = 4096
D = 256
R_PAD = 1024
REP = 1
E = 65536


def _scores_body(relt_ref, w_ref, out_ref):
    s = jnp.sum(relt_ref[...] * w_ref[...], axis=0, keepdims=True)
    out_ref[...] = jnp.broadcast_to(jnp.maximum(s, 0.0) + 1.0, (REP, R_PAD))


def _compute_scores(relt_pad, w_col):
    return pl.pallas_call(
        _scores_body,
        out_shape=jax.ShapeDtypeStruct((REP, R_PAD), jnp.float32),
    )(relt_pad, w_col)


_NC = 2
_NS = 16
_NW = _NC * _NS
_EPW = E // _NW
_E2PW = 2 * _EPW
_DUMP_ROWS = (2 * E + N - 1) // N

_sc_mesh = plsc.VectorSubcoreMesh(core_axis_name="c", subcore_axis_name="s")

_SCATTER_SCRATCH = [
    pltpu.VMEM((_EPW,), jnp.int32),
    pltpu.VMEM((_EPW,), jnp.int32),
    pltpu.VMEM((_EPW,), jnp.int32),
    pltpu.VMEM((_EPW,), jnp.int32),
    pltpu.VMEM((_EPW,), jnp.float32),
    pltpu.VMEM_SHARED((R_PAD,), jnp.float32),
    pltpu.SemaphoreType.DMA,
]


def _stage_table(table_hbm, spmem_ref):
    @pl.when(lax.axis_index("s") == 0)
    def _():
        pltpu.sync_copy(table_hbm, spmem_ref)

    plsc.subcore_barrier()


def _make_phase_kernel(sign):
    @functools.partial(pl.kernel, mesh=_sc_mesh, out_type=(),
                       scratch_types=_SCATTER_SCRATCH)
    def _phase(rows_hbm, cols_hbm, ridx_hbm, table_hbm, buf_ref,
               r_v, c_v, ri_v, flat_v, val_v, tab_s, sem):
        wid = lax.axis_index("s") * _NC + lax.axis_index("c")
        _stage_table(table_hbm, tab_s)
        pltpu.sync_copy(rows_hbm.at[wid], r_v)
        pltpu.sync_copy(cols_hbm.at[wid], c_v)
        pltpu.sync_copy(ridx_hbm.at[wid], ri_v)

        @pl.loop(0, _EPW // 16)
        def _prep(j):
            sl = pl.ds(j * 16, 16)
            flat_v[sl] = r_v[sl] * N + c_v[sl]

        pltpu.async_copy(tab_s.at[ri_v], val_v, sem).wait()

        @pl.loop(0, _EPW // 16)
        def _sgn(j):
            sl = pl.ds(j * 16, 16)
            val_v[sl] = val_v[sl] * sign

        pltpu.async_copy(val_v, buf_ref.at[flat_v], sem).wait()

    return _phase


_sc_fwd = _make_phase_kernel(-1.0)
_sc_bwd = _make_phase_kernel(1.0)

_CHK_SCRATCH = [
    pltpu.VMEM((_E2PW,), jnp.int32),
    pltpu.VMEM((_E2PW,), jnp.float32),
    pltpu.VMEM((_E2PW,), jnp.float32),
    pltpu.VMEM((_E2PW,), jnp.int32),
    pltpu.VMEM((_E2PW,), jnp.float32),
    pltpu.VMEM((_EPW,), jnp.int32),
    pltpu.VMEM((_EPW,), jnp.int32),
    pltpu.VMEM((_EPW,), jnp.int32),
    pltpu.VMEM_SHARED((R_PAD,), jnp.float32),
    pltpu.SemaphoreType.DMA,
    pltpu.SemaphoreType.DMA,
    pltpu.SemaphoreType.DMA,
    pltpu.SemaphoreType.DMA,
]


@functools.partial(pl.kernel, mesh=_sc_mesh, out_type=(),
                   scratch_types=_CHK_SCRATCH)
def _sc_check(rows_hbm, cols_hbm, ridx_hbm, table_hbm, buf_ref,
              pflat_hbm, pval_hbm,
              flat_v, own_v, got_v, pt_v, pv_v, a_v, b_v, ri_v, tab_s,
              sem, semb, semc, semd):
    wid = lax.axis_index("s") * _NC + lax.axis_index("c")
    _stage_table(table_hbm, tab_s)
    pltpu.sync_copy(rows_hbm.at[wid], a_v)
    pltpu.sync_copy(cols_hbm.at[wid], b_v)
    pltpu.sync_copy(ridx_hbm.at[wid], ri_v)

    @pl.loop(0, _EPW // 16)
    def _prep(j):
        sl = pl.ds(j * 16, 16)
        sl2 = pl.ds(_EPW + j * 16, 16)
        flat_v[sl] = a_v[sl] * N + b_v[sl]
        flat_v[sl2] = b_v[sl] * N + a_v[sl]

    cpg = pltpu.async_copy(tab_s.at[ri_v], got_v.at[pl.ds(0, _EPW)], sem)
    cpg.wait()

    @pl.loop(0, _EPW // 16)
    def _own(j):
        sl = pl.ds(j * 16, 16)
        sl2 = pl.ds(_EPW + j * 16, 16)
        t = got_v[sl]
        own_v[sl] = -t
        own_v[sl2] = t

    q = _E2PW // 4
    cps = [
        pltpu.async_copy(buf_ref.at[flat_v.at[pl.ds(i * q, q)]],
                         got_v.at[pl.ds(i * q, q)], s)
        for i, s in enumerate((sem, semb, semc, semd))
    ]
    for cp in cps:
        cp.wait()

    lane = lax.iota(jnp.int32, 16)
    dump = N * N + wid * _E2PW

    @pl.loop(0, _E2PW // 16)
    def _decide(j):
        sl = pl.ds(j * 16, 16)
        g = got_v[sl]
        mine = own_v[sl]
        same_phase = jnp.where(mine < 0.0, g < 0.0, g > 0.0)
        patch = same_phase & (g != mine)
        pt_v[sl] = jnp.where(patch, flat_v[sl], dump + j * 16 + lane)
        pv_v[sl] = (g + mine) * 0.5

    pltpu.sync_copy(pt_v, pflat_hbm.at[wid])
    pltpu.sync_copy(pv_v, pval_hbm.at[wid])


@functools.partial(pl.kernel, mesh=_sc_mesh, out_type=(),
                   scratch_types=[
                       pltpu.VMEM((_E2PW,), jnp.int32),
                       pltpu.VMEM((_E2PW,), jnp.float32),
                       pltpu.SemaphoreType.DMA,
                   ])
def _sc_patch(pflat_hbm, pval_hbm, buf_ref, pt_v, pv_v, sem):
    wid = lax.axis_index("s") * _NC + lax.axis_index("c")
    pltpu.sync_copy(pflat_hbm.at[wid], pt_v)
    pltpu.sync_copy(pval_hbm.at[wid], pv_v)
    pltpu.async_copy(pv_v, buf_ref.at[pt_v], sem).wait()


_BR = 256


def _flash_body(b_ref, adj_ref, inp_ref, bias_ref, out_ref):
    lb = b_ref[...]
    l = jnp.where(lb != 0.0, jnp.abs(lb) - 1.0, 0.0)
    m = l + adj_ref[...]
    mx = jnp.max(m, axis=1, keepdims=True)
    e = jnp.exp(m - mx)
    z = jnp.sum(e, axis=1, keepdims=True)
    acc = jnp.dot(e, inp_ref[...], preferred_element_type=jnp.float32)
    r = acc / z + bias_ref[...]
    out_ref[...] = jnp.where(r > 0.0, r, jnp.exp(r) - 1.0)


def _flash_call(buf, adj, inp, bias_row):
    grid = (N // _BR,)
    return pl.pallas_call(
        _flash_body,
        grid=grid,
        in_specs=[
            pl.BlockSpec((_BR, N), lambda i: (i, 0)),
            pl.BlockSpec((_BR, N), lambda i: (i, 0)),
            pl.BlockSpec((N, D), lambda i: (0, 0)),
            pl.BlockSpec((1, D), lambda i: (0, 0)),
        ],
        out_specs=pl.BlockSpec((_BR, D), lambda i: (i, 0)),
        out_shape=jax.ShapeDtypeStruct((N, D), jnp.float32),
    )(buf, adj, inp, bias_row)


def kernel(input, rel, edge_index, rel_idx, adj, w_rel, bias):
    relt_pad = jnp.pad(rel, ((0, R_PAD - rel.shape[0]), (0, 0))).T
    table = _compute_scores(relt_pad, w_rel.reshape(-1, 1)).reshape(REP * R_PAD)
    e1 = edge_index[0].reshape(_NW, _EPW)
    e2 = edge_index[1].reshape(_NW, _EPW)
    ridx = rel_idx.reshape(_NW, _EPW)
    buf_ref = jax.new_ref(jnp.zeros((N * (N + _DUMP_ROWS),), jnp.float32))
    pflat_ref = jax.new_ref(jnp.zeros((_NW, _E2PW), jnp.int32))
    pval_ref = jax.new_ref(jnp.zeros((_NW, _E2PW), jnp.float32))
    _sc_fwd(e1, e2, ridx, table, buf_ref)
    _sc_bwd(e2, e1, ridx, table, buf_ref)
    _sc_check(e1, e2, ridx, table, buf_ref, pflat_ref, pval_ref)
    _sc_patch(pflat_ref, pval_ref, buf_ref)
    buf = buf_ref[...].reshape(N + _DUMP_ROWS, N)
    return _flash_call(buf, adj, input, bias.reshape(1, D))
